# baseline (speedup 1.0000x reference)
import functools

import jax
import jax.numpy as jnp
from jax.experimental import pallas as pl
from jax.experimental.pallas import tpu as pltpu
from jax.experimental.pallas import tpu_sc as plsc

_ALPH, _D, _T, _L = 4, 1024, 3, 4096
_ROWS, _LANES = 32, 128
_SCALE = 6.0 / (float(_L) * (_L - 1) * (_L - 2))

_HIGH = jax.lax.Precision.HIGHEST

_IDX_B = tuple(t % 4 for t in range(16))
_IDX_A = tuple(4 + t // 4 for t in range(16))


def _iota(shape, dim):
    return jax.lax.broadcasted_iota(jnp.int32, shape, dim)


def _dot(a, b):
    return jax.lax.dot(a, b, precision=_HIGH, preferred_element_type=jnp.float32)


def _dot_t(a, b):
    return jax.lax.dot_general(a, b, (((1,), (1,)), ((), ())),
                               precision=_HIGH, preferred_element_type=jnp.float32)


def _coeff_kernel(seq_ref, v_ref):
    f32 = jnp.float32
    seq = seq_ref[...]

    seqt = jnp.concatenate([seq, seq, seq, seq], axis=0)
    crow = _iota((128, 128), 0) // _ROWS
    XF = (seqt == crow).astype(f32)

    r128, c128 = _iota((128, 128), 0), _iota((128, 128), 1)
    Uexc = (r128 < c128).astype(f32)
    J = jnp.ones((128, 128), f32)
    A128 = ((r128 // _ROWS == c128 // _ROWS) & (c128 < r128)).astype(f32)

    E = _dot(XF, Uexc)
    B = _dot(A128, _dot(XF, J))
    P = E + B

    X4 = XF.reshape(4, 32, 128)
    P4 = P.reshape(4, 32, 128)
    Xa = jnp.broadcast_to(X4[:, None], (4, 4, 32, 128)).reshape(512, 128)
    Pb = jnp.broadcast_to(P4[None, :], (4, 4, 32, 128)).reshape(512, 128)
    YF = Xa * Pb

    r512, c512 = _iota((512, 512), 0), _iota((512, 512), 1)
    A512 = ((r512 // _ROWS == c512 // _ROWS) & (c512 < r512)).astype(f32)
    EY = _dot(YF, Uexc)
    BY = _dot(A512, _dot(YF, J))
    Q = EY + BY

    Qhi = jnp.floor(Q * (1.0 / 4096.0))
    Qlo = Q - Qhi * 4096.0

    Mlo = _dot_t(XF, Qlo)
    Mhi = _dot_t(XF, Qhi)
    diag = ((_iota((128, 512), 0) % _ROWS) ==
            (_iota((128, 512), 1) % _ROWS)).astype(f32)
    Mlo = Mlo * diag
    Mhi = Mhi * diag

    S4 = (_iota((4, 128), 1) // _ROWS == _iota((4, 128), 0)).astype(f32)
    S16 = (_iota((16, 512), 1) // _ROWS == _iota((16, 512), 0)).astype(f32)
    Vlo = _dot_t(_dot(S4, Mlo), S16)
    Vhi = _dot_t(_dot(S4, Mhi), S16)
    v_ref[...] = Vlo + 4096.0 * Vhi


_SC_MESH = plsc.VectorSubcoreMesh(core_axis_name="c", subcore_axis_name="s")


@functools.partial(
    pl.kernel,
    mesh=_SC_MESH,
    out_type=jax.ShapeDtypeStruct((_D,), jnp.float32),
    scratch_types=[
        pltpu.VMEM((64,), jnp.float32),
        pltpu.VMEM((16,), jnp.int32),
        pltpu.VMEM((16,), jnp.int32),
        pltpu.VMEM((32,), jnp.int32),
        pltpu.VMEM((64,), jnp.int32),
        pltpu.VMEM((64,), jnp.float32),
        pltpu.VMEM((_D,), jnp.float32),
        pltpu.VMEM_SHARED((_D,), jnp.float32),
    ],
)
def _scatter_sc(v_hbm, ht_hbm, st_hbm, idx_hbm, out_hbm,
                v_v, ht_v, st_v, idx_v, pos_v, w_v, stage_v, acc_sh):
    wid = jax.lax.axis_index("s") * 2 + jax.lax.axis_index("c")

    @pl.when(wid == 0)
    def _():
        pltpu.sync_copy(v_hbm, v_v)
        pltpu.sync_copy(ht_hbm, ht_v)
        pltpu.sync_copy(st_hbm, st_v)
        pltpu.sync_copy(idx_hbm, idx_v)

        def _zero_stage(j, carry):
            stage_v[pl.ds(j * 16, 16)] = jnp.zeros((16,), jnp.float32)
            return carry

        jax.lax.fori_loop(0, _D // 16, _zero_stage, 0)
        pltpu.sync_copy(stage_v, acc_sh)

        idx_b = idx_v[pl.ds(0, 16)]
        idx_a = idx_v[pl.ds(16, 16)]
        ht_reg = ht_v[pl.ds(0, 16)]
        st_reg = st_v[pl.ds(0, 16)]

        def _vgather(vec, idx):
            dnums = jax.lax.GatherDimensionNumbers(
                offset_dims=(), collapsed_slice_dims=(0,), start_index_map=(0,))
            return jax.lax.gather(
                vec, idx[:, None], dnums, slice_sizes=(1,),
                mode=jax.lax.GatherScatterMode.PROMISE_IN_BOUNDS)

        r1 = _vgather(ht_reg, idx_b)
        r2 = _vgather(ht_reg, idx_a)
        s1 = _vgather(st_reg, idx_b)
        s2 = _vgather(st_reg, idx_a)
        sig12 = (2 * s1 - 1) * (2 * s2 - 1)

        for g in range(4):
            idx_q = jnp.full((16,), 8 + g, jnp.int32)
            r3 = _vgather(ht_reg, idx_q)
            s3 = _vgather(st_reg, idx_q)
            pos_v[pl.ds(16 * g, 16)] = (3 * _D - (r1 + r2 + r3)) % _D
            sig = (sig12 * (2 * s3 - 1)).astype(jnp.float32)
            w_v[pl.ds(16 * g, 16)] = v_v[pl.ds(16 * g, 16)] * sig * _SCALE

        pltpu.sync_copy(w_v, acc_sh.at[pos_v], add=True)
        pltpu.sync_copy(acc_sh, stage_v)
        pltpu.sync_copy(stage_v, out_hbm)


@jax.jit
def kernel(sequence, hash_table, sign_table, Tp0, Tm0):
    del Tp0, Tm0
    seq2d = sequence.reshape(_ROWS, _LANES).astype(jnp.int32)
    ht16 = jnp.zeros((16,), jnp.int32).at[:12].set(
        hash_table.astype(jnp.int32).reshape(12))
    st16 = jnp.zeros((16,), jnp.int32).at[:12].set(
        sign_table.astype(jnp.int32).reshape(12))
    idx32 = jnp.asarray(_IDX_B + _IDX_A, jnp.int32)
    v = pl.pallas_call(
        _coeff_kernel,
        out_shape=jax.ShapeDtypeStruct((4, 16), jnp.float32),
        in_specs=[pl.BlockSpec(memory_space=pltpu.VMEM)],
        out_specs=pl.BlockSpec(memory_space=pltpu.VMEM),
    )(seq2d)
    return _scatter_sc(v.reshape(64), ht16, st16, idx32)


# speedup vs baseline: 1.0785x; 1.0785x over previous
import functools

import jax
import jax.numpy as jnp
from jax.experimental import pallas as pl
from jax.experimental.pallas import tpu as pltpu
from jax.experimental.pallas import tpu_sc as plsc

_ALPH, _D, _T, _L = 4, 1024, 3, 4096
_ROWS, _LANES = 32, 128
_SCALE = 6.0 / (float(_L) * (_L - 1) * (_L - 2))

_HIGH = jax.lax.Precision.HIGHEST

_IDX_B = tuple(t % 4 for t in range(16))
_IDX_A = tuple(4 + t // 4 for t in range(16))


def _iota(shape, dim):
    return jax.lax.broadcasted_iota(jnp.int32, shape, dim)


def _dot(a, b):
    return jax.lax.dot(a, b, precision=_HIGH, preferred_element_type=jnp.float32)


def _dot_t(a, b):
    return jax.lax.dot_general(a, b, (((1,), (1,)), ((), ())),
                               precision=_HIGH, preferred_element_type=jnp.float32)


def _coeff_kernel(seq_ref, v_ref):
    f32 = jnp.float32
    seq = seq_ref[...]

    seqt = jnp.concatenate([seq, seq, seq, seq], axis=0)
    crow = _iota((128, 128), 0) // _ROWS
    XF = (seqt == crow).astype(f32)

    r128, c128 = _iota((128, 128), 0), _iota((128, 128), 1)
    Uexc = (r128 < c128).astype(f32)
    J = jnp.ones((128, 128), f32)
    A128 = ((r128 // _ROWS == c128 // _ROWS) & (c128 < r128)).astype(f32)

    E = _dot(XF, Uexc)
    B = _dot(A128, _dot(XF, J))
    P = E + B

    X4 = XF.reshape(4, 32, 128)
    P4 = P.reshape(4, 32, 128)
    Xa = jnp.broadcast_to(X4[:, None], (4, 4, 32, 128)).reshape(512, 128)
    Pb = jnp.broadcast_to(P4[None, :], (4, 4, 32, 128)).reshape(512, 128)
    YF = Xa * Pb

    r512, c512 = _iota((512, 512), 0), _iota((512, 512), 1)
    A512 = ((r512 // _ROWS == c512 // _ROWS) & (c512 < r512)).astype(f32)
    EY = _dot(YF, Uexc)
    BY = _dot(A512, _dot(YF, J))
    Q = EY + BY

    Qhi = jnp.floor(Q * (1.0 / 4096.0))
    Qlo = Q - Qhi * 4096.0

    Mlo = _dot_t(XF, Qlo)
    Mhi = _dot_t(XF, Qhi)
    diag = ((_iota((128, 512), 0) % _ROWS) ==
            (_iota((128, 512), 1) % _ROWS)).astype(f32)
    Mlo = Mlo * diag
    Mhi = Mhi * diag

    S4 = (_iota((4, 128), 1) // _ROWS == _iota((4, 128), 0)).astype(f32)
    S16 = (_iota((16, 512), 1) // _ROWS == _iota((16, 512), 0)).astype(f32)
    Vlo = _dot_t(_dot(S4, Mlo), S16)
    Vhi = _dot_t(_dot(S4, Mhi), S16)
    v_ref[...] = Vlo + 4096.0 * Vhi


_SC_MESH = plsc.VectorSubcoreMesh(core_axis_name="c", subcore_axis_name="s")


@functools.partial(
    pl.kernel,
    mesh=_SC_MESH,
    out_type=jax.ShapeDtypeStruct((_D,), jnp.float32),
    scratch_types=[
        pltpu.VMEM((64,), jnp.float32),
        pltpu.VMEM((64,), jnp.int32),
        pltpu.VMEM((64,), jnp.int32),
        pltpu.VMEM((64,), jnp.float32),
        pltpu.VMEM((_D,), jnp.float32),
        pltpu.VMEM_SHARED((_D,), jnp.float32),
    ],
)
def _scatter_sc(v_hbm, tbl_hbm, out_hbm, v_v, tbl_v, pos_v, w_v, stage_v, acc_sh):
    wid = jax.lax.axis_index("s") * 2 + jax.lax.axis_index("c")

    @pl.when(wid == 0)
    def _():
        pltpu.sync_copy(v_hbm, v_v)
        pltpu.sync_copy(tbl_hbm, tbl_v)

        def _zero_stage(j, carry):
            stage_v[pl.ds(j * 16, 16)] = jnp.zeros((16,), jnp.float32)
            return carry

        jax.lax.fori_loop(0, _D // 16, _zero_stage, 0)
        pltpu.sync_copy(stage_v, acc_sh)

        ht_reg = tbl_v[pl.ds(0, 16)]
        st_reg = tbl_v[pl.ds(16, 16)]
        idx_b = tbl_v[pl.ds(32, 16)]
        idx_a = tbl_v[pl.ds(48, 16)]

        def _vgather(vec, idx):
            dnums = jax.lax.GatherDimensionNumbers(
                offset_dims=(), collapsed_slice_dims=(0,), start_index_map=(0,))
            return jax.lax.gather(
                vec, idx[:, None], dnums, slice_sizes=(1,),
                mode=jax.lax.GatherScatterMode.PROMISE_IN_BOUNDS)

        r1 = _vgather(ht_reg, idx_b)
        r2 = _vgather(ht_reg, idx_a)
        s1 = _vgather(st_reg, idx_b)
        s2 = _vgather(st_reg, idx_a)
        sig12 = (2 * s1 - 1) * (2 * s2 - 1)

        for g in range(4):
            idx_q = jnp.full((16,), 8 + g, jnp.int32)
            r3 = _vgather(ht_reg, idx_q)
            s3 = _vgather(st_reg, idx_q)
            pos_v[pl.ds(16 * g, 16)] = (3 * _D - (r1 + r2 + r3)) % _D
            sig = (sig12 * (2 * s3 - 1)).astype(jnp.float32)
            w_v[pl.ds(16 * g, 16)] = v_v[pl.ds(16 * g, 16)] * sig * _SCALE

        pltpu.sync_copy(w_v, acc_sh.at[pos_v], add=True)
        pltpu.sync_copy(acc_sh, out_hbm)


@jax.jit
def kernel(sequence, hash_table, sign_table, Tp0, Tm0):
    del Tp0, Tm0
    seq2d = sequence.reshape(_ROWS, _LANES).astype(jnp.int32)
    ht16 = jnp.zeros((16,), jnp.int32).at[:12].set(
        hash_table.astype(jnp.int32).reshape(12))
    st16 = jnp.zeros((16,), jnp.int32).at[:12].set(
        sign_table.astype(jnp.int32).reshape(12))
    idx32 = jnp.asarray(_IDX_B + _IDX_A, jnp.int32)
    tbl = jnp.concatenate([ht16, st16, idx32])
    v = pl.pallas_call(
        _coeff_kernel,
        out_shape=jax.ShapeDtypeStruct((4, 16), jnp.float32),
        in_specs=[pl.BlockSpec(memory_space=pltpu.VMEM)],
        out_specs=pl.BlockSpec(memory_space=pltpu.VMEM),
    )(seq2d)
    return _scatter_sc(v.reshape(64), tbl)


# speedup vs baseline: 1.1569x; 1.0726x over previous
import functools

import jax
import jax.numpy as jnp
from jax.experimental import pallas as pl
from jax.experimental.pallas import tpu as pltpu
from jax.experimental.pallas import tpu_sc as plsc

_ALPH, _D, _T, _L = 4, 1024, 3, 4096
_SCALE = 6.0 / (float(_L) * (_L - 1) * (_L - 2))

_NCHUNK = 16
_CHUNK = _L // _NCHUNK
_NSTAT = 96

_IDX_B = tuple(t % 4 for t in range(16))
_IDX_A = tuple(4 + t // 4 for t in range(16))


def _vgather(vec, idx):
    dnums = jax.lax.GatherDimensionNumbers(
        offset_dims=(), collapsed_slice_dims=(0,), start_index_map=(0,))
    return jax.lax.gather(
        vec, idx[:, None], dnums, slice_sizes=(1,),
        mode=jax.lax.GatherScatterMode.PROMISE_IN_BOUNDS)


_SC_MESH = plsc.VectorSubcoreMesh(core_axis_name="c", subcore_axis_name="s")


@functools.partial(
    pl.kernel,
    mesh=_SC_MESH,
    out_type=jax.ShapeDtypeStruct((_D,), jnp.float32),
    scratch_types=[
        pltpu.VMEM((_CHUNK,), jnp.int32),
        pltpu.VMEM((64,), jnp.int32),
        pltpu.VMEM((_NSTAT,), jnp.float32),
        pltpu.VMEM((_NCHUNK * _NSTAT,), jnp.float32),
        pltpu.VMEM((64,), jnp.int32),
        pltpu.VMEM((64,), jnp.float32),
        pltpu.VMEM((_D,), jnp.float32),
        pltpu.VMEM_SHARED((_NCHUNK * _NSTAT,), jnp.float32),
        pltpu.VMEM_SHARED((_D,), jnp.float32),
    ],
)
def _sketch_sc(seq_hbm, tbl_hbm, out_hbm,
               seq_v, tbl_v, stat_v, all_v, pos_v, w_v, stage_v,
               stats_sh, acc_sh):
    f32 = jnp.float32
    cid = jax.lax.axis_index("c")
    sid = jax.lax.axis_index("s")

    @pl.when(cid == 0)
    def _():
        pltpu.sync_copy(tbl_hbm, tbl_v)
        pltpu.sync_copy(seq_hbm.at[pl.ds(sid * _CHUNK, _CHUNK)], seq_v)

        patb = tbl_v[pl.ds(32, 16)]
        pata = tbl_v[pl.ds(48, 16)] - 4
        patbf = patb.astype(f32)
        pataf = pata.astype(f32)
        one = jnp.float32(1.0)
        zero = jnp.zeros((16,), f32)

        def _chunk_step(g, carry):
            lp, lq, c0, c1, c2, c3 = carry
            creg = seq_v[pl.ds(g * 16, 16)]
            for l in range(16):
                cvec = _vgather(creg, jnp.full((16,), l, jnp.int32))
                cf = cvec.astype(f32)
                e0 = one - jnp.minimum(jnp.abs(cf - 0.0), one)
                e1 = one - jnp.minimum(jnp.abs(cf - 1.0), one)
                e2 = one - jnp.minimum(jnp.abs(cf - 2.0), one)
                e3 = one - jnp.minimum(jnp.abs(cf - 3.0), one)
                ea = one - jnp.minimum(jnp.abs(pataf - cf), one)
                eb = one - jnp.minimum(jnp.abs(patbf - cf), one)
                c0 = c0 + e0 * lq
                c1 = c1 + e1 * lq
                c2 = c2 + e2 * lq
                c3 = c3 + e3 * lq
                lq = lq + ea * lp
                lp = lp + eb
            return lp, lq, c0, c1, c2, c3

        lp, lq, c0, c1, c2, c3 = jax.lax.fori_loop(
            0, _CHUNK // 16, _chunk_step, (zero,) * 6)

        stat_v[pl.ds(0, 16)] = lp
        stat_v[pl.ds(16, 16)] = lq
        stat_v[pl.ds(32, 16)] = c0
        stat_v[pl.ds(48, 16)] = c1
        stat_v[pl.ds(64, 16)] = c2
        stat_v[pl.ds(80, 16)] = c3
        pltpu.sync_copy(stat_v, stats_sh.at[pl.ds(sid * _NSTAT, _NSTAT)])

        @pl.when(sid == 0)
        def _():
            def _zero_stage(j, carry):
                stage_v[pl.ds(j * 16, 16)] = jnp.zeros((16,), f32)
                return carry
            jax.lax.fori_loop(0, _D // 16, _zero_stage, 0)
            pltpu.sync_copy(stage_v, acc_sh)

        plsc.subcore_barrier()

        @pl.when(sid == 0)
        def _():
            pltpu.sync_copy(stats_sh, all_v)

            n16 = jnp.zeros((16,), f32)
            w16 = jnp.zeros((16,), f32)
            v = [jnp.zeros((16,), f32) for _ in range(4)]
            for t in range(_NCHUNK):
                base = t * _NSTAT
                lcnt = all_v[pl.ds(base, 16)]
                a_ab = all_v[pl.ds(base + 16, 16)]
                for q in range(4):
                    cq = all_v[pl.ds(base + 32 + 16 * q, 16)]
                    bq = _vgather(a_ab, pata + 4 * q)
                    lcq = _vgather(lcnt, jnp.full((16,), q, jnp.int32))
                    v[q] = v[q] + w16 * lcq + bq * n16 + cq
                w16 = w16 + _vgather(lcnt, pata) * n16 + a_ab
                n16 = n16 + lcnt

            ht_reg = tbl_v[pl.ds(0, 16)]
            st_reg = tbl_v[pl.ds(16, 16)]
            idx_b = tbl_v[pl.ds(32, 16)]
            idx_a = tbl_v[pl.ds(48, 16)]
            r1 = _vgather(ht_reg, idx_b)
            r2 = _vgather(ht_reg, idx_a)
            s1 = _vgather(st_reg, idx_b)
            s2 = _vgather(st_reg, idx_a)
            sig12 = (2 * s1 - 1) * (2 * s2 - 1)

            for g in range(4):
                idx_q = jnp.full((16,), 8 + g, jnp.int32)
                r3 = _vgather(ht_reg, idx_q)
                s3 = _vgather(st_reg, idx_q)
                pos_v[pl.ds(16 * g, 16)] = (3 * _D - (r1 + r2 + r3)) % _D
                sig = (sig12 * (2 * s3 - 1)).astype(f32)
                w_v[pl.ds(16 * g, 16)] = v[g] * sig * _SCALE

            pltpu.sync_copy(w_v, acc_sh.at[pos_v], add=True)
            pltpu.sync_copy(acc_sh, out_hbm)


@jax.jit
def kernel(sequence, hash_table, sign_table, Tp0, Tm0):
    del Tp0, Tm0
    seq = sequence.astype(jnp.int32)
    ht16 = jnp.zeros((16,), jnp.int32).at[:12].set(
        hash_table.astype(jnp.int32).reshape(12))
    st16 = jnp.zeros((16,), jnp.int32).at[:12].set(
        sign_table.astype(jnp.int32).reshape(12))
    idx32 = jnp.asarray(_IDX_B + _IDX_A, jnp.int32)
    tbl = jnp.concatenate([ht16, st16, idx32])
    return _sketch_sc(seq, tbl)


# speedup vs baseline: 1.1890x; 1.0278x over previous
import functools

import jax
import jax.numpy as jnp
from jax.experimental import pallas as pl
from jax.experimental.pallas import tpu as pltpu
from jax.experimental.pallas import tpu_sc as plsc

_ALPH, _D, _T, _L = 4, 1024, 3, 4096
_SCALE = 6.0 / (float(_L) * (_L - 1) * (_L - 2))

_NCHUNK = 16
_CHUNK = _L // _NCHUNK
_NSTAT = 96

_IDX_B = tuple(t % 4 for t in range(16))
_IDX_A = tuple(4 + t // 4 for t in range(16))


def _vgather(vec, idx):
    dnums = jax.lax.GatherDimensionNumbers(
        offset_dims=(), collapsed_slice_dims=(0,), start_index_map=(0,))
    return jax.lax.gather(
        vec, idx[:, None], dnums, slice_sizes=(1,),
        mode=jax.lax.GatherScatterMode.PROMISE_IN_BOUNDS)


_SC_MESH = plsc.VectorSubcoreMesh(core_axis_name="c", subcore_axis_name="s")


@functools.partial(
    pl.kernel,
    mesh=_SC_MESH,
    out_type=jax.ShapeDtypeStruct((_D,), jnp.float32),
    scratch_types=[
        pltpu.VMEM((_CHUNK,), jnp.int32),
        pltpu.VMEM((64,), jnp.int32),
        pltpu.VMEM((_NSTAT,), jnp.float32),
        pltpu.VMEM((_NCHUNK * _NSTAT,), jnp.float32),
        pltpu.VMEM((64,), jnp.int32),
        pltpu.VMEM((64,), jnp.float32),
        pltpu.VMEM((_D,), jnp.float32),
        pltpu.VMEM_SHARED((_NCHUNK * _NSTAT,), jnp.float32),
        pltpu.VMEM_SHARED((_D,), jnp.float32),
        pltpu.SemaphoreType.DMA,
    ],
)
def _sketch_sc(seq_hbm, tbl_hbm, out_hbm,
               seq_v, tbl_v, stat_v, all_v, pos_v, w_v, stage_v,
               stats_sh, acc_sh, sem):
    f32 = jnp.float32
    cid = jax.lax.axis_index("c")
    sid = jax.lax.axis_index("s")

    @pl.when(cid == 0)
    def _():
        seq_dma = pltpu.async_copy(
            seq_hbm.at[pl.ds(sid * _CHUNK, _CHUNK)], seq_v, sem)
        pltpu.sync_copy(tbl_hbm, tbl_v)

        def _zero_stage(j, carry):
            stage_v[pl.ds(j * 16, 16)] = jnp.zeros((16,), f32)
            return carry
        jax.lax.fori_loop(0, 4, _zero_stage, 0)
        pltpu.sync_copy(stage_v.at[pl.ds(0, 64)],
                        acc_sh.at[pl.ds(sid * 64, 64)])
        seq_dma.wait()

        patb = tbl_v[pl.ds(32, 16)]
        pata = tbl_v[pl.ds(48, 16)] - 4
        patbf = patb.astype(f32)
        pataf = pata.astype(f32)
        one = jnp.float32(1.0)
        zero = jnp.zeros((16,), f32)

        def _chunk_step(g, carry):
            lp, lq, c0, c1, c2, c3 = carry
            creg = seq_v[pl.ds(g * 16, 16)]
            for l in range(16):
                cvec = _vgather(creg, jnp.full((16,), l, jnp.int32))
                cf = cvec.astype(f32)
                e0 = one - jnp.minimum(jnp.abs(cf - 0.0), one)
                e1 = one - jnp.minimum(jnp.abs(cf - 1.0), one)
                e2 = one - jnp.minimum(jnp.abs(cf - 2.0), one)
                e3 = one - jnp.minimum(jnp.abs(cf - 3.0), one)
                ea = one - jnp.minimum(jnp.abs(pataf - cf), one)
                eb = one - jnp.minimum(jnp.abs(patbf - cf), one)
                c0 = c0 + e0 * lq
                c1 = c1 + e1 * lq
                c2 = c2 + e2 * lq
                c3 = c3 + e3 * lq
                lq = lq + ea * lp
                lp = lp + eb
            return lp, lq, c0, c1, c2, c3

        lp, lq, c0, c1, c2, c3 = jax.lax.fori_loop(
            0, _CHUNK // 16, _chunk_step, (zero,) * 6)

        stat_v[pl.ds(0, 16)] = lp
        stat_v[pl.ds(16, 16)] = lq
        stat_v[pl.ds(32, 16)] = c0
        stat_v[pl.ds(48, 16)] = c1
        stat_v[pl.ds(64, 16)] = c2
        stat_v[pl.ds(80, 16)] = c3
        pltpu.sync_copy(stat_v, stats_sh.at[pl.ds(sid * _NSTAT, _NSTAT)])

        plsc.subcore_barrier()

        @pl.when(sid == 0)
        def _():
            pltpu.sync_copy(stats_sh, all_v)

            n16 = jnp.zeros((16,), f32)
            w16 = jnp.zeros((16,), f32)
            v = [jnp.zeros((16,), f32) for _ in range(4)]
            for t in range(_NCHUNK):
                base = t * _NSTAT
                lcnt = all_v[pl.ds(base, 16)]
                a_ab = all_v[pl.ds(base + 16, 16)]
                for q in range(4):
                    cq = all_v[pl.ds(base + 32 + 16 * q, 16)]
                    bq = _vgather(a_ab, pata + 4 * q)
                    lcq = _vgather(lcnt, jnp.full((16,), q, jnp.int32))
                    v[q] = v[q] + w16 * lcq + bq * n16 + cq
                w16 = w16 + _vgather(lcnt, pata) * n16 + a_ab
                n16 = n16 + lcnt

            ht_reg = tbl_v[pl.ds(0, 16)]
            st_reg = tbl_v[pl.ds(16, 16)]
            idx_b = tbl_v[pl.ds(32, 16)]
            idx_a = tbl_v[pl.ds(48, 16)]
            r1 = _vgather(ht_reg, idx_b)
            r2 = _vgather(ht_reg, idx_a)
            s1 = _vgather(st_reg, idx_b)
            s2 = _vgather(st_reg, idx_a)
            sig12 = (2 * s1 - 1) * (2 * s2 - 1)

            for g in range(4):
                idx_q = jnp.full((16,), 8 + g, jnp.int32)
                r3 = _vgather(ht_reg, idx_q)
                s3 = _vgather(st_reg, idx_q)
                pos_v[pl.ds(16 * g, 16)] = (3 * _D - (r1 + r2 + r3)) % _D
                sig = (sig12 * (2 * s3 - 1)).astype(f32)
                w_v[pl.ds(16 * g, 16)] = v[g] * sig * _SCALE

            pltpu.sync_copy(w_v, acc_sh.at[pos_v], add=True)
            pltpu.sync_copy(acc_sh, out_hbm)


@jax.jit
def kernel(sequence, hash_table, sign_table, Tp0, Tm0):
    del Tp0, Tm0
    seq = sequence.astype(jnp.int32)
    ht16 = jnp.zeros((16,), jnp.int32).at[:12].set(
        hash_table.astype(jnp.int32).reshape(12))
    st16 = jnp.zeros((16,), jnp.int32).at[:12].set(
        sign_table.astype(jnp.int32).reshape(12))
    idx32 = jnp.asarray(_IDX_B + _IDX_A, jnp.int32)
    tbl = jnp.concatenate([ht16, st16, idx32])
    return _sketch_sc(seq, tbl)
